# baseline (speedup 1.0000x reference)
import jax
import jax.numpy as jnp
from jax.experimental import pallas as pl
from jax.experimental.pallas import tpu as pltpu

BATCH = 16384
VOCAB = 1000
EMBED = 16
BLOCK_M = 2048
STREAMS = 8
SUB_M = BLOCK_M // STREAMS


def _matmul_body(*refs):
    x_refs = refs[:STREAMS]
    w_ref = refs[STREAMS]
    o_ref = refs[STREAMS + 1]
    wb = w_ref[...].astype(jnp.bfloat16)
    for j in range(STREAMS):
        xb = x_refs[j][...].astype(jnp.bfloat16)
        o_ref[j * SUB_M:(j + 1) * SUB_M, :] = jax.lax.dot_general(
            xb, wb,
            dimension_numbers=(((1,), (0,)), ((), ())),
            preferred_element_type=jnp.float32,
            precision=jax.lax.Precision.DEFAULT,
        )


def kernel(one_hot, weight):
    grid = (BATCH // BLOCK_M,)
    in_specs = [
        pl.BlockSpec((SUB_M, VOCAB), lambda i, j=j: (i * STREAMS + j, 0))
        for j in range(STREAMS)
    ] + [pl.BlockSpec((VOCAB, EMBED), lambda i: (0, 0))]
    return pl.pallas_call(
        _matmul_body,
        grid=grid,
        in_specs=in_specs,
        out_specs=pl.BlockSpec((BLOCK_M, EMBED), lambda i: (i, 0)),
        out_shape=jax.ShapeDtypeStruct((BATCH, EMBED), jnp.float32),
        compiler_params=pltpu.CompilerParams(
            dimension_semantics=("arbitrary",),
        ),
    )(*([one_hot] * STREAMS), weight)
